# baseline (speedup 1.0000x reference)
import dataclasses
import functools

import jax
import jax.numpy as jnp
from jax import lax
from jax.experimental import pallas as pl
from jax.experimental.pallas import tpu as pltpu
from jax.experimental.pallas import tpu_sc as plsc

E = 320000
D = 128
NW = 32
PER_W = E // NW
C = 80
STEPS = PER_W // C
PAIRS = (STEPS - 1) // 2
LANES = 16
assert PER_W % C == 0 and C % LANES == 0 and STEPS % 2 == 1


def _make_kernel():
    mesh = plsc.VectorSubcoreMesh(core_axis_name="c", subcore_axis_name="s")
    cp = pltpu.CompilerParams()
    if "needs_layout_passes" in pltpu.CompilerParams.__dataclass_fields__:
        cp = dataclasses.replace(cp, needs_layout_passes=False)
    cp = dataclasses.replace(cp, use_tc_tiling_on_sc=False)

    @functools.partial(
        pl.kernel,
        mesh=mesh,
        compiler_params=cp,
        out_type=jax.ShapeDtypeStruct((E,), jnp.float32),
        scratch_types=[
            pltpu.VMEM((PER_W,), jnp.int32),
            pltpu.VMEM((PER_W,), jnp.int32),
            pltpu.VMEM((C, D // 2), jnp.int32),
            pltpu.VMEM((C, D // 2), jnp.int32),
            pltpu.VMEM((C, D // 2), jnp.int32),
            pltpu.VMEM((C, D // 2), jnp.int32),
            pltpu.VMEM((PER_W,), jnp.float32),
            pltpu.VMEM((C, LANES), jnp.float32),
            pltpu.SemaphoreType.DMA,
            pltpu.SemaphoreType.DMA,
            pltpu.SemaphoreType.DMA,
            pltpu.SemaphoreType.DMA,
        ],
    )
    def k(xh_hbm, xt_hbm, ei_hbm, out_hbm,
          i0_v, i1_v, h0_v, t0_v, h1_v, t1_v, o_v, sq_v,
          sem_h0, sem_t0, sem_h1, sem_t1):
        wid = lax.axis_index("s") * 2 + lax.axis_index("c")
        base = wid * PER_W
        pltpu.sync_copy(ei_hbm.at[pl.ds(base, PER_W)], i0_v)
        pltpu.sync_copy(ei_hbm.at[pl.ds(E + base, PER_W)], i1_v)

        iota = lax.iota(jnp.int32, LANES)
        cols = [jnp.full((LANES,), j, jnp.int32) for j in range(LANES)]
        hi_mask = jnp.full((LANES,), -65536, jnp.int32)
        sixteen = jnp.full((LANES,), 16, jnp.int32)

        def start(s, h_ref, t_ref, sh, st):
            off = s * C
            pltpu.async_copy(xh_hbm.at[i0_v.at[pl.ds(off, C)]], h_ref, sh)
            pltpu.async_copy(xt_hbm.at[i1_v.at[pl.ds(off, C)]], t_ref, st)

        def wait(s, h_ref, t_ref, sh, st):
            off = s * C
            pltpu.make_async_copy(
                xh_hbm.at[i0_v.at[pl.ds(off, C)]], h_ref, sh).wait()
            pltpu.make_async_copy(
                xt_hbm.at[i1_v.at[pl.ds(off, C)]], t_ref, st).wait()

        def compute(s, h_ref, t_ref):
            off = s * C

            @pl.loop(0, C)
            def _(e):
                acc = None
                for c in range(D // (2 * LANES)):
                    hv = plsc.bitcast(
                        h_ref[e, pl.ds(c * LANES, LANES)], jnp.bfloat16)
                    tv = plsc.bitcast(
                        t_ref[e, pl.ds(c * LANES, LANES)], jnp.bfloat16)
                    w = plsc.bitcast(hv * tv, jnp.int32)
                    p_hi = plsc.bitcast(w & hi_mask, jnp.float32)
                    p_lo = plsc.bitcast(
                        lax.shift_left(w, sixteen), jnp.float32)
                    acc = p_hi + p_lo if acc is None else acc + p_hi + p_lo
                sq_v[e, :] = acc

            @pl.loop(0, C // LANES)
            def _(g):
                rows = g * LANES + iota
                ovec = plsc.load_gather(sq_v, [rows, cols[0]])
                for j in range(1, LANES):
                    ovec = ovec + plsc.load_gather(sq_v, [rows, cols[j]])
                o_v[pl.ds(off + g * LANES, LANES)] = ovec

        start(0, h0_v, t0_v, sem_h0, sem_t0)

        @pl.loop(0, PAIRS)
        def _(p):
            s0 = 2 * p
            start(s0 + 1, h1_v, t1_v, sem_h1, sem_t1)
            wait(s0, h0_v, t0_v, sem_h0, sem_t0)
            compute(s0, h0_v, t0_v)
            start(s0 + 2, h0_v, t0_v, sem_h0, sem_t0)
            wait(s0 + 1, h1_v, t1_v, sem_h1, sem_t1)
            compute(s0 + 1, h1_v, t1_v)

        wait(STEPS - 1, h0_v, t0_v, sem_h0, sem_t0)
        compute(STEPS - 1, h0_v, t0_v)

        pltpu.sync_copy(o_v, out_hbm.at[pl.ds(base, PER_W)])

    return k


_sc_kernel = _make_kernel()


def _pack_bf16_words(x):
    v = lax.bitcast_convert_type(x, jnp.uint32)
    r = (v + 0x7FFF + ((v >> 16) & 1)) >> 16
    w = r[:, : D // 2] | (r[:, D // 2:] << 16)
    return lax.bitcast_convert_type(w, jnp.int32)


def kernel(x_head, x_tail, edge_label_index):
    ei = edge_label_index.astype(jnp.int32).reshape(-1)
    return _sc_kernel(_pack_bf16_words(x_head), _pack_bf16_words(x_tail), ei)


# speedup vs baseline: 1.0239x; 1.0239x over previous
import dataclasses
import functools

import jax
import jax.numpy as jnp
from jax import lax
from jax.experimental import pallas as pl
from jax.experimental.pallas import tpu as pltpu
from jax.experimental.pallas import tpu_sc as plsc

E = 320000
D = 128
NW = 32
PER_W = E // NW
C = 80
STEPS = PER_W // C
PAIRS = (STEPS - 1) // 2
LANES = 16
assert PER_W % C == 0 and C % LANES == 0 and STEPS % 2 == 1


def _make_kernel():
    mesh = plsc.VectorSubcoreMesh(core_axis_name="c", subcore_axis_name="s")
    cp = pltpu.CompilerParams()
    if "needs_layout_passes" in pltpu.CompilerParams.__dataclass_fields__:
        cp = dataclasses.replace(cp, needs_layout_passes=False)

    @functools.partial(
        pl.kernel,
        mesh=mesh,
        compiler_params=cp,
        out_type=jax.ShapeDtypeStruct((E,), jnp.float32),
        scratch_types=[
            pltpu.VMEM((PER_W,), jnp.int32),
            pltpu.VMEM((PER_W,), jnp.int32),
            pltpu.VMEM((C, D), jnp.float32),
            pltpu.VMEM((C, D), jnp.float32),
            pltpu.VMEM((C, D), jnp.float32),
            pltpu.VMEM((C, D), jnp.float32),
            pltpu.VMEM((PER_W,), jnp.float32),
            pltpu.VMEM((C, LANES), jnp.float32),
            pltpu.SemaphoreType.DMA,
            pltpu.SemaphoreType.DMA,
            pltpu.SemaphoreType.DMA,
            pltpu.SemaphoreType.DMA,
        ],
    )
    def k(xh_hbm, xt_hbm, ei_hbm, out_hbm,
          i0_v, i1_v, h0_v, t0_v, h1_v, t1_v, o_v, sq_v,
          sem_h0, sem_t0, sem_h1, sem_t1):
        wid = lax.axis_index("s") * 2 + lax.axis_index("c")
        base = wid * PER_W
        pltpu.sync_copy(ei_hbm.at[pl.ds(base, PER_W)], i0_v)
        pltpu.sync_copy(ei_hbm.at[pl.ds(E + base, PER_W)], i1_v)

        iota = lax.iota(jnp.int32, LANES)
        cols = [jnp.full((LANES,), j, jnp.int32) for j in range(LANES)]

        def start(s, h_ref, t_ref, sh, st):
            off = s * C
            pltpu.async_copy(xh_hbm.at[i0_v.at[pl.ds(off, C)]], h_ref, sh)
            pltpu.async_copy(xt_hbm.at[i1_v.at[pl.ds(off, C)]], t_ref, st)

        def wait(s, h_ref, t_ref, sh, st):
            off = s * C
            pltpu.make_async_copy(
                xh_hbm.at[i0_v.at[pl.ds(off, C)]], h_ref, sh).wait()
            pltpu.make_async_copy(
                xt_hbm.at[i1_v.at[pl.ds(off, C)]], t_ref, st).wait()

        def compute(s, h_ref, t_ref):
            off = s * C

            @pl.loop(0, C)
            def _(e):
                acc = h_ref[e, pl.ds(0, LANES)] * t_ref[e, pl.ds(0, LANES)]
                for c in range(1, D // LANES):
                    acc = acc + (h_ref[e, pl.ds(c * LANES, LANES)]
                                 * t_ref[e, pl.ds(c * LANES, LANES)])
                sq_v[e, :] = acc

            @pl.loop(0, C // LANES)
            def _(g):
                rows = g * LANES + iota
                ovec = plsc.load_gather(sq_v, [rows, cols[0]])
                for j in range(1, LANES):
                    ovec = ovec + plsc.load_gather(sq_v, [rows, cols[j]])
                o_v[pl.ds(off + g * LANES, LANES)] = ovec

        start(0, h0_v, t0_v, sem_h0, sem_t0)

        @pl.loop(0, PAIRS)
        def _(p):
            s0 = 2 * p
            start(s0 + 1, h1_v, t1_v, sem_h1, sem_t1)
            wait(s0, h0_v, t0_v, sem_h0, sem_t0)
            compute(s0, h0_v, t0_v)
            start(s0 + 2, h0_v, t0_v, sem_h0, sem_t0)
            wait(s0 + 1, h1_v, t1_v, sem_h1, sem_t1)
            compute(s0 + 1, h1_v, t1_v)

        wait(STEPS - 1, h0_v, t0_v, sem_h0, sem_t0)
        compute(STEPS - 1, h0_v, t0_v)

        pltpu.sync_copy(o_v, out_hbm.at[pl.ds(base, PER_W)])

    return k


_sc_kernel = _make_kernel()


def kernel(x_head, x_tail, edge_label_index):
    ei = edge_label_index.astype(jnp.int32).reshape(-1)
    return _sc_kernel(x_head, x_tail, ei)
